# Initial kernel scaffold; baseline (speedup 1.0000x reference)
import jax
import jax.numpy as jnp
from jax.experimental import pallas as pl


def kernel(nodes, batch, edges, batch_edges, graph_globals, W1, b1, W2, b2):
    raise NotImplementedError("write your pallas kernel here")


# speedup vs baseline: 5.3387x; 5.3387x over previous
import functools

import jax
import jax.numpy as jnp
from jax import lax
from jax.experimental import pallas as pl
from jax.experimental.pallas import tpu as pltpu
from jax.experimental.pallas import tpu_sc as plsc

N_GRAPHS = 512
N_NODES = 100000
N_EDGES = 1600000
NODE_DIM = 128
EDGE_DIM = 16
GLOBAL_DIM = 128
HIDDEN = 64

NC = 2
NS = 16
NW = NC * NS

PACK = NODE_DIM // EDGE_DIM
N_SUPER = N_EDGES // PACK
TRASH = N_GRAPHS
MAX_IMPURE = NW * 16

N_CHUNK = 80
N_CHUNKS = N_NODES // N_CHUNK
N_CHUNKS_PER_CORE = N_CHUNKS // NC
N_ITERS = (N_CHUNKS_PER_CORE + NS - 1) // NS

E_CHUNK = 80
E_CHUNKS = N_SUPER // E_CHUNK
E_CHUNKS_PER_CORE = E_CHUNKS // NC
E_ITERS = (E_CHUNKS_PER_CORE + NS - 1) // NS


def _seg_sums_sc(nodes, batch_i32, e8, sidx, imp, imp_ids, zn, ze):
    mesh = plsc.VectorSubcoreMesh(core_axis_name="c", subcore_axis_name="s")

    @functools.partial(
        pl.kernel,
        out_type=(
            jax.ShapeDtypeStruct((NC, N_GRAPHS, NODE_DIM), jnp.float32),
            jax.ShapeDtypeStruct((NC, N_GRAPHS, NODE_DIM), jnp.float32),
        ),
        mesh=mesh,
        scratch_types=[
            pltpu.VMEM_SHARED((N_GRAPHS, NODE_DIM), jnp.float32),
            pltpu.VMEM_SHARED((N_GRAPHS + 1, NODE_DIM), jnp.float32),
            pltpu.VMEM((N_CHUNK,), jnp.int32),
            pltpu.VMEM((N_CHUNK, NODE_DIM), jnp.float32),
            pltpu.VMEM((E_CHUNK,), jnp.int32),
            pltpu.VMEM((E_CHUNK, NODE_DIM), jnp.float32),
            pltpu.VMEM((16,), jnp.int32),
            pltpu.VMEM((16, 16), jnp.int32),
            pltpu.VMEM((16,), jnp.int32),
            pltpu.VMEM((1, NODE_DIM), jnp.float32),
            pltpu.VMEM((16, NODE_DIM), jnp.float32),
        ],
    )
    def k(nodes_hbm, batch_hbm, e8_hbm, sidx_hbm, imp_hbm, impid_hbm,
          zn_hbm, ze_hbm, outn_hbm, oute_hbm,
          acc_n, acc_e, idx_n, nbuf, idx_e, ebuf,
          impbuf, impidbuf, cidx, corrbuf, crows):
        c = lax.axis_index("c")
        s = lax.axis_index("s")
        w = s * NC + c

        @pl.when(s == 0)
        def _():
            pltpu.sync_copy(zn_hbm, acc_n)
            pltpu.sync_copy(ze_hbm, acc_e)

        for r in range(16):
            for b in range(PACK):
                crows[r, pl.ds(b * 16, 16)] = jnp.zeros((16,), jnp.float32)

        plsc.subcore_barrier()

        def nbody(i, carry):
            cc = s + i * NS
            @pl.when(cc < N_CHUNKS_PER_CORE)
            def _():
                base = (c * N_CHUNKS_PER_CORE + cc) * N_CHUNK
                pltpu.sync_copy(batch_hbm.at[pl.ds(base, N_CHUNK)], idx_n)
                pltpu.sync_copy(nodes_hbm.at[pl.ds(base, N_CHUNK)], nbuf)
                pltpu.sync_copy(nbuf, acc_n.at[idx_n], add=True)
            return carry

        lax.fori_loop(0, N_ITERS, nbody, 0)

        def ebody(i, carry):
            cc = s + i * NS
            @pl.when(cc < E_CHUNKS_PER_CORE)
            def _():
                base = (c * E_CHUNKS_PER_CORE + cc) * E_CHUNK
                pltpu.sync_copy(sidx_hbm.at[pl.ds(base, E_CHUNK)], idx_e)
                pltpu.sync_copy(e8_hbm.at[pl.ds(base, E_CHUNK)], ebuf)
                pltpu.sync_copy(ebuf, acc_e.at[idx_e], add=True)
            return carry

        lax.fori_loop(0, E_ITERS, ebody, 0)

        pltpu.sync_copy(imp_hbm.at[pl.ds(w * 16, 16)], impbuf)
        pltpu.sync_copy(impid_hbm.at[pl.ds(w * 16, 16)], impidbuf)
        impvec = impbuf[...]
        for t in range(16):
            j = impvec[t]
            @pl.when(j < N_SUPER)
            def _():
                pltpu.sync_copy(e8_hbm.at[pl.ds(j, 1)], corrbuf)
                cidx[...] = impidbuf[t]
                for b in range(PACK):
                    crows[b, pl.ds(b * 16, 16)] = corrbuf[0, pl.ds(b * 16, 16)]
                pltpu.sync_copy(crows, acc_e.at[cidx], add=True)

        plsc.subcore_barrier()

        @pl.when(s == 0)
        def _():
            pltpu.sync_copy(acc_n, outn_hbm.at[c])
            pltpu.sync_copy(acc_e.at[pl.ds(0, N_GRAPHS)], oute_hbm.at[c])

    return k(nodes, batch_i32, e8, sidx, imp, imp_ids, zn, ze)


def _mlp_body(gg, npart, epart, w1g, w1n, w1e, b1r, w2, b2r, out):
    ns = npart[0] + npart[1]
    e128 = epart[0] + epart[1]
    es = e128[:, 0:16]
    for b in range(1, PACK):
        es = es + e128[:, b * 16:(b + 1) * 16]
    x = (
        jnp.dot(gg[...], w1g[...], preferred_element_type=jnp.float32)
        + jnp.dot(ns, w1n[...], preferred_element_type=jnp.float32)
        + jnp.dot(es, w1e[...], preferred_element_type=jnp.float32)
        + b1r[...]
    )
    h = jnp.maximum(x, 0.0)
    out[...] = jnp.dot(h, w2[...], preferred_element_type=jnp.float32) + b2r[...]


def kernel(nodes, batch, edges, batch_edges, graph_globals, W1, b1, W2, b2):
    batch_i32 = batch.astype(jnp.int32)
    be_i32 = batch_edges.astype(jnp.int32)

    e8 = edges.reshape(N_SUPER, NODE_DIM)

    heads = be_i32[::PACK]
    nxt = jnp.concatenate([heads[1:], jnp.full((1,), -1, jnp.int32)])
    pure = heads == nxt
    sidx = jnp.where(pure, heads, TRASH).astype(jnp.int32)
    imp = jnp.nonzero(~pure, size=MAX_IMPURE, fill_value=N_SUPER)[0].astype(jnp.int32)
    lanes = jnp.arange(16, dtype=jnp.int32)[None, :]
    gpos = jnp.clip(imp[:, None] * PACK + lanes, 0, N_EDGES - 1)
    valid = (lanes < PACK) & (imp[:, None] < N_SUPER)
    imp_ids = jnp.where(valid, be_i32[gpos], TRASH).astype(jnp.int32)

    zn = jnp.zeros((N_GRAPHS, NODE_DIM), jnp.float32)
    ze = jnp.zeros((N_GRAPHS + 1, NODE_DIM), jnp.float32)

    npart, epart = _seg_sums_sc(nodes, batch_i32, e8, sidx, imp, imp_ids, zn, ze)

    w1g = W1[:GLOBAL_DIM]
    w1n = W1[GLOBAL_DIM:GLOBAL_DIM + NODE_DIM]
    w1e = W1[GLOBAL_DIM + NODE_DIM:]
    b1r = b1.reshape(1, HIDDEN)
    b2r = b2.reshape(1, GLOBAL_DIM)

    out = pl.pallas_call(
        _mlp_body,
        out_shape=jax.ShapeDtypeStruct((N_GRAPHS, GLOBAL_DIM), jnp.float32),
    )(graph_globals, npart, epart, w1g, w1n, w1e, b1r, W2, b2r)
    return out


# speedup vs baseline: 5.4138x; 1.0141x over previous
import functools

import jax
import jax.numpy as jnp
from jax import lax
from jax.experimental import pallas as pl
from jax.experimental.pallas import tpu as pltpu
from jax.experimental.pallas import tpu_sc as plsc

N_GRAPHS = 512
N_NODES = 100000
N_EDGES = 1600000
NODE_DIM = 128
EDGE_DIM = 16
GLOBAL_DIM = 128
HIDDEN = 64

NC = 2
NS = 16

N_CHUNK = 80
N_CHUNKS = N_NODES // N_CHUNK
N_CHUNKS_PER_CORE = N_CHUNKS // NC
N_ITERS = (N_CHUNKS_PER_CORE + NS - 1) // NS

E_CHUNK = 640
E_CHUNKS = N_EDGES // E_CHUNK
E_CHUNKS_PER_CORE = E_CHUNKS // NC
E_ITERS = (E_CHUNKS_PER_CORE + NS - 1) // NS
E_GROUPS = E_CHUNK // 64

EROWS = N_GRAPHS // 8


def _seg_sums_sc(nodes, batch_i32, edges, be_i32, zn, ze):
    mesh = plsc.VectorSubcoreMesh(core_axis_name="c", subcore_axis_name="s")

    @functools.partial(
        pl.kernel,
        out_type=(
            jax.ShapeDtypeStruct((NC, N_GRAPHS, NODE_DIM), jnp.float32),
            jax.ShapeDtypeStruct((NC, EROWS, NODE_DIM), jnp.float32),
        ),
        mesh=mesh,
        scratch_types=[
            pltpu.VMEM_SHARED((N_GRAPHS, NODE_DIM), jnp.float32),
            pltpu.VMEM_SHARED((EROWS, NODE_DIM), jnp.float32),
            pltpu.VMEM((N_CHUNK,), jnp.int32),
            pltpu.VMEM((N_CHUNK, NODE_DIM), jnp.float32),
            pltpu.VMEM((E_CHUNK,), jnp.int32),
            pltpu.VMEM((E_CHUNK, EDGE_DIM), jnp.float32),
            pltpu.VMEM((EROWS, NODE_DIM), jnp.float32),
            pltpu.VMEM((EROWS,), jnp.int32),
        ],
    )
    def k(nodes_hbm, batch_hbm, e_hbm, be_hbm, zn_hbm, ze_hbm,
          outn_hbm, oute_hbm,
          acc_n, acc_e, idx_n, nbuf, idwin, ebuf, eacc, idv):
        c = lax.axis_index("c")
        s = lax.axis_index("s")
        lane = jnp.arange(16, dtype=jnp.int32)

        @pl.when(s == 0)
        def _():
            pltpu.sync_copy(zn_hbm, acc_n)
            pltpu.sync_copy(ze_hbm, acc_e)

        for r in range(EROWS):
            for b in range(8):
                eacc[r, pl.ds(b * 16, 16)] = jnp.zeros((16,), jnp.float32)
        for kk in range(EROWS // 16):
            idv[pl.ds(16 * kk, 16)] = lane + 16 * kk

        plsc.subcore_barrier()

        def nbody(i, carry):
            cc = s + i * NS
            @pl.when(cc < N_CHUNKS_PER_CORE)
            def _():
                base = (c * N_CHUNKS_PER_CORE + cc) * N_CHUNK
                pltpu.sync_copy(batch_hbm.at[pl.ds(base, N_CHUNK)], idx_n)
                pltpu.sync_copy(nodes_hbm.at[pl.ds(base, N_CHUNK)], nbuf)
                pltpu.sync_copy(nbuf, acc_n.at[idx_n], add=True)
            return carry

        lax.fori_loop(0, N_ITERS, nbody, 0)

        def eadd(seg, vec):
            row = seg // 8
            col = (seg % 8) * 16
            eacc[row, pl.ds(col, 16)] = eacc[row, pl.ds(col, 16)] + vec

        def ebody(i, carry):
            cc = s + i * NS
            @pl.when(cc < E_CHUNKS_PER_CORE)
            def _():
                eoff = (c * E_CHUNKS_PER_CORE + cc) * E_CHUNK
                pltpu.sync_copy(be_hbm.at[pl.ds(eoff, E_CHUNK)], idwin)
                pltpu.sync_copy(e_hbm.at[pl.ds(eoff, E_CHUNK)], ebuf)

                def gbody(g, gcarry):
                    base = g * 64
                    v = [idwin[pl.ds(base + 16 * kk, 16)] for kk in range(4)]
                    s0 = v[0][0]
                    s63 = v[3][15]

                    @pl.when(s0 == s63)
                    def _():
                        a0 = ebuf[base] + ebuf[base + 1]
                        a1 = ebuf[base + 2] + ebuf[base + 3]
                        for r in range(4, 64, 4):
                            a0 = a0 + (ebuf[base + r] + ebuf[base + r + 1])
                            a1 = a1 + (ebuf[base + r + 2] + ebuf[base + r + 3])
                        eadd(s0, a0 + a1)

                    @pl.when(s0 != s63)
                    def _():
                        for kk in range(4):
                            sk0 = v[kk][0]
                            sk15 = v[kk][15]
                            sub = base + 16 * kk

                            @pl.when(sk0 == sk15)
                            def _():
                                b0 = ebuf[sub] + ebuf[sub + 1]
                                b1 = ebuf[sub + 2] + ebuf[sub + 3]
                                for r in range(4, 16, 4):
                                    b0 = b0 + (ebuf[sub + r] + ebuf[sub + r + 1])
                                    b1 = b1 + (ebuf[sub + r + 2] + ebuf[sub + r + 3])
                                eadd(sk0, b0 + b1)

                            @pl.when(sk0 != sk15)
                            def _():
                                for r in range(16):
                                    eadd(v[kk][r], ebuf[sub + r])
                    return gcarry

                lax.fori_loop(0, E_GROUPS, gbody, 0)
            return carry

        lax.fori_loop(0, E_ITERS, ebody, 0)

        pltpu.sync_copy(eacc, acc_e.at[idv], add=True)

        plsc.subcore_barrier()

        @pl.when(s == 0)
        def _():
            pltpu.sync_copy(acc_n, outn_hbm.at[c])
            pltpu.sync_copy(acc_e, oute_hbm.at[c])

    return k(nodes, batch_i32, edges, be_i32, zn, ze)


def _mlp_body(gg, ns0, ns1, es, w1g, w1n, w1e, b1r, w2, b2r, out):
    ns = ns0[...] + ns1[...]
    x = (
        jnp.dot(gg[...], w1g[...], preferred_element_type=jnp.float32)
        + jnp.dot(ns, w1n[...], preferred_element_type=jnp.float32)
        + jnp.dot(es[...], w1e[...], preferred_element_type=jnp.float32)
        + b1r[...]
    )
    h = jnp.maximum(x, 0.0)
    out[...] = jnp.dot(h, w2[...], preferred_element_type=jnp.float32) + b2r[...]


def kernel(nodes, batch, edges, batch_edges, graph_globals, W1, b1, W2, b2):
    batch_i32 = batch.astype(jnp.int32)
    be_i32 = batch_edges.astype(jnp.int32)

    zn = jnp.zeros((N_GRAPHS, NODE_DIM), jnp.float32)
    ze = jnp.zeros((EROWS, NODE_DIM), jnp.float32)

    npart, epart = _seg_sums_sc(nodes, batch_i32, edges, be_i32, zn, ze)

    es = (epart[0] + epart[1]).reshape(N_GRAPHS, EDGE_DIM)

    w1g = W1[:GLOBAL_DIM]
    w1n = W1[GLOBAL_DIM:GLOBAL_DIM + NODE_DIM]
    w1e = W1[GLOBAL_DIM + NODE_DIM:]
    b1r = b1.reshape(1, HIDDEN)
    b2r = b2.reshape(1, GLOBAL_DIM)

    out = pl.pallas_call(
        _mlp_body,
        out_shape=jax.ShapeDtypeStruct((N_GRAPHS, GLOBAL_DIM), jnp.float32),
    )(graph_globals, npart[0], npart[1], es, w1g, w1n, w1e, b1r, W2, b2r)
    return out
